# Initial kernel scaffold; baseline (speedup 1.0000x reference)
import jax
import jax.numpy as jnp
from jax.experimental import pallas as pl


def kernel(h, edge_index, rel_ids, w_relation):
    raise NotImplementedError("write your pallas kernel here")


# speedup vs baseline: 1.2429x; 1.2429x over previous
import jax
import jax.numpy as jnp
from jax import lax
from jax.experimental import pallas as pl
from jax.experimental.pallas import tpu as pltpu
from jax.experimental.pallas import tpu_sc as plsc

N_NODES = 10000
N_EDGES = 320000
H_DIM = 128
NUM_RELS = 8

NC = 2
NS = 16
L = 16
NW = NC * NS
EPW = N_EDGES // NW
B = 80
NB = EPW // B
NG = B // L
NCHUNK = H_DIM // L


def _sc_body(u_hbm, v_hbm, r_hbm, w_hbm, h_hbm, out_hbm,
             uidx, vidx, ridx, srows, drows, rrows, colbuf, score, sem):
    wid = lax.axis_index("s") * NC + lax.axis_index("c")
    base0 = wid * EPW
    lane = lax.iota(jnp.int32, L)

    def batch_body(b, carry):
        base = base0 + b * B
        pltpu.sync_copy(u_hbm.at[pl.ds(base, B)], uidx)
        pltpu.sync_copy(v_hbm.at[pl.ds(base, B)], vidx)
        pltpu.sync_copy(r_hbm.at[pl.ds(base, B)], ridx)
        cp_s = pltpu.async_copy(h_hbm.at[uidx], srows, sem)
        cp_d = pltpu.async_copy(h_hbm.at[vidx], drows, sem)
        cp_r = pltpu.async_copy(w_hbm.at[ridx], rrows, sem)
        cp_s.wait()
        cp_d.wait()
        cp_r.wait()

        def group_body(g, carry2):
            for j in range(L):
                e = g * L + j
                acc = jnp.zeros((L,), jnp.float32)
                for c in range(NCHUNK):
                    s = srows[e, pl.ds(c * L, L)]
                    t = drows[e, pl.ds(c * L, L)]
                    w = rrows[e, pl.ds(c * L, L)]
                    acc = acc + s * t * w
                colbuf[pl.ds(j * L, L)] = acc
            sc = jnp.zeros((L,), jnp.float32)
            for i in range(L):
                sc = sc + plsc.load_gather(colbuf, [lane * L + i])
            score[pl.ds(g * L, L)] = sc
            return carry2

        lax.fori_loop(0, NG, group_body, 0)
        pltpu.sync_copy(score, out_hbm.at[pl.ds(base, B)])
        return carry

    lax.fori_loop(0, NB, batch_body, 0)


def kernel(h, edge_index, rel_ids, w_relation):
    u = edge_index[0].astype(jnp.int32)
    v = edge_index[1].astype(jnp.int32)
    r = rel_ids.astype(jnp.int32)
    run = pl.kernel(
        _sc_body,
        mesh=plsc.VectorSubcoreMesh(core_axis_name="c", subcore_axis_name="s"),
        compiler_params=pltpu.CompilerParams(needs_layout_passes=False),
        out_type=jax.ShapeDtypeStruct((N_EDGES,), jnp.float32),
        scratch_types=[
            pltpu.VMEM((B,), jnp.int32),
            pltpu.VMEM((B,), jnp.int32),
            pltpu.VMEM((B,), jnp.int32),
            pltpu.VMEM((B, H_DIM), jnp.float32),
            pltpu.VMEM((B, H_DIM), jnp.float32),
            pltpu.VMEM((B, H_DIM), jnp.float32),
            pltpu.VMEM((L * L,), jnp.float32),
            pltpu.VMEM((B,), jnp.float32),
            pltpu.SemaphoreType.DMA,
        ],
    )
    return run(u, v, r, w_relation.astype(jnp.float32), h.astype(jnp.float32))


# speedup vs baseline: 1.2546x; 1.0094x over previous
import jax
import jax.numpy as jnp
from jax import lax
from jax.experimental import pallas as pl
from jax.experimental.pallas import tpu as pltpu
from jax.experimental.pallas import tpu_sc as plsc

N_NODES = 10000
N_EDGES = 320000
H_DIM = 128
NUM_RELS = 8

NC = 2
NS = 16
L = 16
NW = NC * NS
EPW = N_EDGES // NW
B = 80
NB = EPW // B
NG = B // L
NCHUNK = H_DIM // L
NPAIR = (NB - 1) // 2


def _sc_body(idx_hbm, w_hbm, h_hbm, out_hbm,
             i0, i1, sr0, dr0, rr0, sr1, dr1, rr1,
             colbuf, score, sem0, sem1):
    wid = lax.axis_index("s") * NC + lax.axis_index("c")
    bid0 = wid * NB
    lane = lax.iota(jnp.int32, L)

    def issue(i_ref, sr, dr, rr, sem):
        pltpu.async_copy(h_hbm.at[i_ref.at[0]], sr, sem)
        pltpu.async_copy(h_hbm.at[i_ref.at[1]], dr, sem)
        pltpu.async_copy(w_hbm.at[i_ref.at[2]], rr, sem)

    def drain(i_ref, sr, dr, rr, sem):
        pltpu.make_async_copy(h_hbm.at[i_ref.at[0]], sr, sem).wait()
        pltpu.make_async_copy(h_hbm.at[i_ref.at[1]], dr, sem).wait()
        pltpu.make_async_copy(w_hbm.at[i_ref.at[2]], rr, sem).wait()

    def compute(bofs, sr, dr, rr):
        def group_body(g, carry):
            for j in range(L):
                e = g * L + j
                acc = jnp.zeros((L,), jnp.float32)
                for c in range(NCHUNK):
                    s = sr[e, pl.ds(c * L, L)]
                    t = dr[e, pl.ds(c * L, L)]
                    w = rr[e, pl.ds(c * L, L)]
                    acc = acc + s * t * w
                colbuf[pl.ds(j * L, L)] = acc
            sc = jnp.zeros((L,), jnp.float32)
            for i in range(L):
                sc = sc + plsc.load_gather(colbuf, [lane * L + i])
            score[pl.ds(bofs * B + g * L, L)] = sc
            return carry

        lax.fori_loop(0, NG, group_body, 0)

    pltpu.sync_copy(idx_hbm.at[bid0], i0)
    issue(i0, sr0, dr0, rr0, sem0)
    pltpu.sync_copy(idx_hbm.at[bid0 + 1], i1)

    def pair_body(p, carry):
        b0 = 2 * p
        issue(i1, sr1, dr1, rr1, sem1)
        drain(i0, sr0, dr0, rr0, sem0)
        pltpu.sync_copy(idx_hbm.at[bid0 + b0 + 2], i0)
        compute(b0, sr0, dr0, rr0)
        issue(i0, sr0, dr0, rr0, sem0)
        drain(i1, sr1, dr1, rr1, sem1)

        @pl.when(b0 + 3 < NB)
        def _():
            pltpu.sync_copy(idx_hbm.at[bid0 + b0 + 3], i1)

        compute(b0 + 1, sr1, dr1, rr1)
        return carry

    lax.fori_loop(0, NPAIR, pair_body, 0)

    drain(i0, sr0, dr0, rr0, sem0)
    compute(NB - 1, sr0, dr0, rr0)

    pltpu.sync_copy(score, out_hbm.at[pl.ds(wid * EPW, EPW)])


def kernel(h, edge_index, rel_ids, w_relation):
    u = edge_index[0].astype(jnp.int32)
    v = edge_index[1].astype(jnp.int32)
    r = rel_ids.astype(jnp.int32)
    idx3 = (jnp.stack([u, v, r], axis=0)
            .reshape(3, NW * NB, B)
            .transpose(1, 0, 2))
    run = pl.kernel(
        _sc_body,
        mesh=plsc.VectorSubcoreMesh(core_axis_name="c", subcore_axis_name="s"),
        compiler_params=pltpu.CompilerParams(needs_layout_passes=False),
        out_type=jax.ShapeDtypeStruct((N_EDGES,), jnp.float32),
        scratch_types=[
            pltpu.VMEM((3, B), jnp.int32),
            pltpu.VMEM((3, B), jnp.int32),
            pltpu.VMEM((B, H_DIM), jnp.float32),
            pltpu.VMEM((B, H_DIM), jnp.float32),
            pltpu.VMEM((B, H_DIM), jnp.float32),
            pltpu.VMEM((B, H_DIM), jnp.float32),
            pltpu.VMEM((B, H_DIM), jnp.float32),
            pltpu.VMEM((B, H_DIM), jnp.float32),
            pltpu.VMEM((L * L,), jnp.float32),
            pltpu.VMEM((EPW,), jnp.float32),
            pltpu.SemaphoreType.DMA,
            pltpu.SemaphoreType.DMA,
        ],
    )
    return run(idx3, w_relation.astype(jnp.float32), h.astype(jnp.float32))


# speedup vs baseline: 6.8087x; 5.4268x over previous
import jax
import jax.numpy as jnp
from jax import lax
from jax.experimental import pallas as pl
from jax.experimental.pallas import tpu as pltpu
from jax.experimental.pallas import tpu_sc as plsc

N_NODES = 10000
N_EDGES = 320000
H_DIM = 128
NUM_RELS = 8

NC = 2
NS = 16
L = 16
NW = NC * NS
EPW = N_EDGES // NW
B = 80
NB = EPW // B
NG = B // L
NCHUNK = H_DIM // L
NPAIR = (NB - 1) // 2


def _sc_body(idx_hbm, w_hbm, h_hbm, out_hbm,
             i0, i1, rc0, rc1, sr0, dr0, sr1, dr1,
             wv, colbuf, score, sem0, sem1):
    wid = lax.axis_index("s") * NC + lax.axis_index("c")
    bid0 = wid * NB
    lane = lax.iota(jnp.int32, L)

    pltpu.sync_copy(w_hbm, wv)

    def issue(i_ref, sr, dr, sem):
        pltpu.async_copy(h_hbm.at[i_ref.at[0]], sr, sem)
        pltpu.async_copy(h_hbm.at[i_ref.at[1]], dr, sem)

    def drain(i_ref, rc, sr, dr, sem):
        pltpu.make_async_copy(h_hbm.at[i_ref.at[0]], sr, sem).wait()
        pltpu.make_async_copy(h_hbm.at[i_ref.at[1]], dr, sem).wait()
        for k in range(NG):
            rc[pl.ds(k * L, L)] = i_ref[2, pl.ds(k * L, L)]

    def compute(bofs, rc, sr, dr):
        def group_body(g, carry):
            for j in range(L):
                e = g * L + j
                rsp = plsc.load_gather(rc, [jnp.full((L,), e, jnp.int32)])
                widx = rsp * H_DIM + lane
                acc = jnp.zeros((L,), jnp.float32)
                for c in range(NCHUNK):
                    s = sr[e, pl.ds(c * L, L)]
                    t = dr[e, pl.ds(c * L, L)]
                    w = plsc.load_gather(wv, [widx + c * L])
                    acc = acc + s * t * w
                colbuf[pl.ds(j * L, L)] = acc
            sc = jnp.zeros((L,), jnp.float32)
            for i in range(L):
                sc = sc + plsc.load_gather(colbuf, [lane * L + i])
            score[pl.ds(bofs * B + g * L, L)] = sc
            return carry

        lax.fori_loop(0, NG, group_body, 0)

    pltpu.sync_copy(idx_hbm.at[bid0], i0)
    issue(i0, sr0, dr0, sem0)
    pltpu.sync_copy(idx_hbm.at[bid0 + 1], i1)

    def pair_body(p, carry):
        b0 = 2 * p
        issue(i1, sr1, dr1, sem1)
        drain(i0, rc0, sr0, dr0, sem0)
        pltpu.sync_copy(idx_hbm.at[bid0 + b0 + 2], i0)
        compute(b0, rc0, sr0, dr0)
        issue(i0, sr0, dr0, sem0)
        drain(i1, rc1, sr1, dr1, sem1)

        @pl.when(b0 + 3 < NB)
        def _():
            pltpu.sync_copy(idx_hbm.at[bid0 + b0 + 3], i1)

        compute(b0 + 1, rc1, sr1, dr1)
        return carry

    lax.fori_loop(0, NPAIR, pair_body, 0)

    drain(i0, rc0, sr0, dr0, sem0)
    compute(NB - 1, rc0, sr0, dr0)

    pltpu.sync_copy(score, out_hbm.at[pl.ds(wid * EPW, EPW)])


def kernel(h, edge_index, rel_ids, w_relation):
    u = edge_index[0].astype(jnp.int32)
    v = edge_index[1].astype(jnp.int32)
    r = rel_ids.astype(jnp.int32)
    idx3 = (jnp.stack([u, v, r], axis=0)
            .reshape(3, NW * NB, B)
            .transpose(1, 0, 2))
    run = pl.kernel(
        _sc_body,
        mesh=plsc.VectorSubcoreMesh(core_axis_name="c", subcore_axis_name="s"),
        compiler_params=pltpu.CompilerParams(needs_layout_passes=False),
        out_type=jax.ShapeDtypeStruct((N_EDGES,), jnp.float32),
        scratch_types=[
            pltpu.VMEM((3, B), jnp.int32),
            pltpu.VMEM((3, B), jnp.int32),
            pltpu.VMEM((B,), jnp.int32),
            pltpu.VMEM((B,), jnp.int32),
            pltpu.VMEM((B, H_DIM), jnp.float32),
            pltpu.VMEM((B, H_DIM), jnp.float32),
            pltpu.VMEM((B, H_DIM), jnp.float32),
            pltpu.VMEM((B, H_DIM), jnp.float32),
            pltpu.VMEM((NUM_RELS * H_DIM,), jnp.float32),
            pltpu.VMEM((L * L,), jnp.float32),
            pltpu.VMEM((EPW,), jnp.float32),
            pltpu.SemaphoreType.DMA,
            pltpu.SemaphoreType.DMA,
        ],
    )
    return run(idx3, w_relation.reshape(-1).astype(jnp.float32),
               h.astype(jnp.float32))


# speedup vs baseline: 7.3651x; 1.0817x over previous
import jax
import jax.numpy as jnp
from jax import lax
from jax.experimental import pallas as pl
from jax.experimental.pallas import tpu as pltpu
from jax.experimental.pallas import tpu_sc as plsc

N_NODES = 10000
N_EDGES = 320000
H_DIM = 128
NUM_RELS = 8

NC = 2
NS = 16
L = 16
NW = NC * NS
EPW = N_EDGES // NW
B = 80
NB = EPW // B
NG = B // L
NCHUNK = H_DIM // L
NPAIR = (NB - 1) // 2


def _sc_body(idx_hbm, w_hbm, h_hbm, out_hbm,
             i0, i1, rc0, rc1, sr0, dr0, sr1, dr1,
             wv, colbuf, score, sem0, sem1):
    wid = lax.axis_index("s") * NC + lax.axis_index("c")
    bid0 = wid * NB
    lane = lax.iota(jnp.int32, L)

    pltpu.sync_copy(w_hbm, wv)

    def issue(i_ref, sr, dr, sem):
        pltpu.async_copy(h_hbm.at[i_ref.at[0]], sr, sem)
        pltpu.async_copy(h_hbm.at[i_ref.at[1]], dr, sem)

    def drain(i_ref, rc, sr, dr, sem):
        pltpu.make_async_copy(h_hbm.at[i_ref.at[0]], sr, sem).wait()
        pltpu.make_async_copy(h_hbm.at[i_ref.at[1]], dr, sem).wait()
        for k in range(NG):
            rc[pl.ds(k * L, L)] = i_ref[2, pl.ds(k * L, L)]

    def compute(bofs, rc, sr, dr):
        def group_body(g, carry):
            e0 = g * L
            rvg = rc[pl.ds(e0, L)]
            rsp0 = plsc.load_gather(rc, [jnp.full((L,), e0, jnp.int32)])
            nmix = jnp.sum(jnp.where(rvg != rsp0, 1, 0))

            @pl.when(nmix == 0)
            def _fast():
                wrow = [plsc.load_gather(wv, [rsp0 * H_DIM + c * L + lane])
                        for c in range(NCHUNK)]
                for j in range(L):
                    e = e0 + j
                    acc = jnp.zeros((L,), jnp.float32)
                    for c in range(NCHUNK):
                        s = sr[e, pl.ds(c * L, L)]
                        t = dr[e, pl.ds(c * L, L)]
                        acc = acc + s * t * wrow[c]
                    colbuf[pl.ds(j * L, L)] = acc

            @pl.when(nmix != 0)
            def _slow():
                for j in range(L):
                    e = e0 + j
                    rsp = plsc.load_gather(rc, [jnp.full((L,), e, jnp.int32)])
                    widx = rsp * H_DIM + lane
                    acc = jnp.zeros((L,), jnp.float32)
                    for c in range(NCHUNK):
                        s = sr[e, pl.ds(c * L, L)]
                        t = dr[e, pl.ds(c * L, L)]
                        w = plsc.load_gather(wv, [widx + c * L])
                        acc = acc + s * t * w
                    colbuf[pl.ds(j * L, L)] = acc
            sc = jnp.zeros((L,), jnp.float32)
            for i in range(L):
                sc = sc + plsc.load_gather(colbuf, [lane * L + i])
            score[pl.ds(bofs * B + g * L, L)] = sc
            return carry

        lax.fori_loop(0, NG, group_body, 0)

    pltpu.sync_copy(idx_hbm.at[bid0], i0)
    issue(i0, sr0, dr0, sem0)
    pltpu.sync_copy(idx_hbm.at[bid0 + 1], i1)

    def pair_body(p, carry):
        b0 = 2 * p
        issue(i1, sr1, dr1, sem1)
        drain(i0, rc0, sr0, dr0, sem0)
        pltpu.sync_copy(idx_hbm.at[bid0 + b0 + 2], i0)
        compute(b0, rc0, sr0, dr0)
        issue(i0, sr0, dr0, sem0)
        drain(i1, rc1, sr1, dr1, sem1)

        @pl.when(b0 + 3 < NB)
        def _():
            pltpu.sync_copy(idx_hbm.at[bid0 + b0 + 3], i1)

        compute(b0 + 1, rc1, sr1, dr1)
        return carry

    lax.fori_loop(0, NPAIR, pair_body, 0)

    drain(i0, rc0, sr0, dr0, sem0)
    compute(NB - 1, rc0, sr0, dr0)

    pltpu.sync_copy(score, out_hbm.at[pl.ds(wid * EPW, EPW)])


def kernel(h, edge_index, rel_ids, w_relation):
    u = edge_index[0].astype(jnp.int32)
    v = edge_index[1].astype(jnp.int32)
    r = rel_ids.astype(jnp.int32)
    idx3 = (jnp.stack([u, v, r], axis=0)
            .reshape(3, NW * NB, B)
            .transpose(1, 0, 2))
    run = pl.kernel(
        _sc_body,
        mesh=plsc.VectorSubcoreMesh(core_axis_name="c", subcore_axis_name="s"),
        compiler_params=pltpu.CompilerParams(needs_layout_passes=False),
        out_type=jax.ShapeDtypeStruct((N_EDGES,), jnp.float32),
        scratch_types=[
            pltpu.VMEM((3, B), jnp.int32),
            pltpu.VMEM((3, B), jnp.int32),
            pltpu.VMEM((B,), jnp.int32),
            pltpu.VMEM((B,), jnp.int32),
            pltpu.VMEM((B, H_DIM), jnp.float32),
            pltpu.VMEM((B, H_DIM), jnp.float32),
            pltpu.VMEM((B, H_DIM), jnp.float32),
            pltpu.VMEM((B, H_DIM), jnp.float32),
            pltpu.VMEM((NUM_RELS * H_DIM,), jnp.float32),
            pltpu.VMEM((L * L,), jnp.float32),
            pltpu.VMEM((EPW,), jnp.float32),
            pltpu.SemaphoreType.DMA,
            pltpu.SemaphoreType.DMA,
        ],
    )
    return run(idx3, w_relation.reshape(-1).astype(jnp.float32),
               h.astype(jnp.float32))
